# baseline (speedup 1.0000x reference)
import functools

import jax
import jax.numpy as jnp
from jax import lax
from jax.experimental import pallas as pl
from jax.experimental.pallas import tpu as pltpu
from jax.experimental.pallas import tpu_sc as plsc

B = 16384
D = 64
H = 64
V = 1000000
SHALF = 253952
NWIN = 4

NC = 2
NS = 16
NW = NC * NS
B_PER_W = B // NW
CHUNK = 128
NCHUNK = B_PER_W // CHUNK


_PCOLS = 4096
_NBLK = SHALF // _PCOLS
_LASTBLK = (V - 1) // _PCOLS


def _proj_body(t0_ref, t1_ref, t2_ref, t3_ref, w_ref, b_ref, out_ref):
    w = w_ref[...]
    bias = b_ref[...]
    for k, t_ref in enumerate((t0_ref, t1_ref, t2_ref, t3_ref)):
        a = lax.dot_general(
            t_ref[...], w,
            dimension_numbers=(((0,), (1,)), ((), ())),
            preferred_element_type=jnp.float32,
        )
        ai = lax.bitcast_convert_type(a + bias, jnp.int32)
        ar = ai + 0x8000
        lo = lax.shift_right_logical(ar[:, 0:32], 16)
        hi = ar[:, 32:64] & jnp.int32(-65536)
        out_ref[:, 32 * k:32 * (k + 1)] = hi | lo


def _win_spec(k):
    return pl.BlockSpec(
        (D, _PCOLS),
        lambda g, _k=k: (0, jnp.minimum(g + _NBLK * _k, _LASTBLK)),
    )


def _tc_project(tT, W, b2):
    return pl.pallas_call(
        _proj_body,
        grid=(_NBLK,),
        in_specs=[_win_spec(k) for k in range(NWIN)] + [
            pl.BlockSpec((H, D), lambda g: (0, 0)),
            pl.BlockSpec((1, H), lambda g: (0, 0)),
        ],
        out_specs=pl.BlockSpec((_PCOLS, 4 * 32), lambda g: (g, 0)),
        out_shape=jax.ShapeDtypeStruct((SHALF, 4 * 32), jnp.int32),
    )(tT, tT, tT, tT, W, b2)


def _sc_gather(p2i, idx3):
    mesh = plsc.VectorSubcoreMesh(core_axis_name="c", subcore_axis_name="s")

    @functools.partial(
        pl.kernel,
        out_type=jax.ShapeDtypeStruct((B, 128), jnp.int32),
        mesh=mesh,
        scratch_types=[
            pltpu.VMEM((NCHUNK, CHUNK), jnp.int32),
            pltpu.VMEM((B_PER_W, 128), jnp.int32),
            pltpu.SemaphoreType.DMA,
        ],
    )
    def gather_kernel(p_hbm, idx_hbm, out_hbm, idx_v, rows_v, sem):
        wid = lax.axis_index("s") * NC + lax.axis_index("c")
        base = wid * B_PER_W
        pltpu.sync_copy(idx_hbm.at[wid], idx_v)
        copies = []
        for j in range(NCHUNK):
            copies.append(
                pltpu.async_copy(
                    p_hbm.at[idx_v.at[j]],
                    rows_v.at[pl.ds(j * CHUNK, CHUNK)],
                    sem,
                )
            )
        for c in copies:
            c.wait()
        pltpu.sync_copy(rows_v, out_hbm.at[pl.ds(base, B_PER_W)])

    return gather_kernel(p2i, idx3)


_SROWS = 2048


def _sel_body(q_ref, m_ref, out_ref):
    q = q_ref[...]
    idx = m_ref[...]
    w01 = jnp.where(idx < SHALF, q[:, 0:32], q[:, 32:64])
    w23 = jnp.where(idx < 3 * SHALF, q[:, 64:96], q[:, 96:128])
    w = jnp.where(idx < 2 * SHALF, w01, w23)
    out_ref[:, 0:32] = lax.bitcast_convert_type(
        lax.shift_left(w, 16), jnp.float32)
    out_ref[:, 32:64] = lax.bitcast_convert_type(
        w & jnp.int32(-65536), jnp.float32)


def _tc_select(q, idxcol):
    grid = (B // _SROWS,)
    return pl.pallas_call(
        _sel_body,
        grid=grid,
        in_specs=[
            pl.BlockSpec((_SROWS, 128), lambda i: (i, 0)),
            pl.BlockSpec((_SROWS, 1), lambda i: (i, 0)),
        ],
        out_specs=pl.BlockSpec((_SROWS, H), lambda i: (i, 0)),
        out_shape=jax.ShapeDtypeStruct((B, H), jnp.float32),
    )(q, idxcol)


def kernel(item_embeds, table, W, b):
    idx = item_embeds.astype(jnp.int32)
    k = (idx >= SHALF).astype(jnp.int32) + (idx >= 2 * SHALF) + (idx >= 3 * SHALF)
    m = idx - k * SHALF
    idx3 = m.reshape(NW, NCHUNK, CHUNK)
    tT = table.T
    p2i = _tc_project(tT, W, b.reshape(1, H))
    q = _sc_gather(p2i, idx3)
    return _tc_select(q, idx.reshape(B, 1))


# speedup vs baseline: 1.6985x; 1.6985x over previous
import functools

import jax
import jax.numpy as jnp
from jax import lax
from jax.experimental import pallas as pl
from jax.experimental.pallas import tpu as pltpu
from jax.experimental.pallas import tpu_sc as plsc

B = 16384
D = 64
H = 64
V = 1000000
S = 507904

NC = 2
NS = 16
NW = NC * NS
B_PER_W = B // NW
CHUNK = 128
NCHUNK = B_PER_W // CHUNK


_PCOLS = 16384


def _proj_body(ta_ref, tb_ref, w_ref, b_ref, out_ref):
    w = w_ref[...]
    bias = b_ref[...]
    a1 = lax.dot_general(
        ta_ref[...], w,
        dimension_numbers=(((0,), (1,)), ((), ())),
        preferred_element_type=jnp.float32,
    )
    a2 = lax.dot_general(
        tb_ref[...], w,
        dimension_numbers=(((0,), (1,)), ((), ())),
        preferred_element_type=jnp.float32,
    )
    out_ref[:, 0:H] = a1 + bias
    out_ref[:, H:2 * H] = a2 + bias


def _tc_project(tT, W, b2):
    grid = (S // _PCOLS,)
    return pl.pallas_call(
        _proj_body,
        grid=grid,
        in_specs=[
            pl.BlockSpec((D, _PCOLS), lambda g: (0, g)),
            pl.BlockSpec(
                (D, _PCOLS),
                lambda g: (0, jnp.minimum(g + S // _PCOLS, (V - 1) // _PCOLS)),
            ),
            pl.BlockSpec((H, D), lambda g: (0, 0)),
            pl.BlockSpec((1, H), lambda g: (0, 0)),
        ],
        out_specs=pl.BlockSpec((_PCOLS, 2 * H), lambda g: (g, 0)),
        out_shape=jax.ShapeDtypeStruct((S, 2 * H), jnp.float32),
    )(tT, tT, W, b2)


def _sc_gather(p2, idx3):
    mesh = plsc.VectorSubcoreMesh(core_axis_name="c", subcore_axis_name="s")

    @functools.partial(
        pl.kernel,
        out_type=jax.ShapeDtypeStruct((B, 2 * H), jnp.float32),
        mesh=mesh,
        scratch_types=[
            pltpu.VMEM((NCHUNK, CHUNK), jnp.int32),
            pltpu.VMEM((B_PER_W, 2 * H), jnp.float32),
            pltpu.SemaphoreType.DMA,
        ],
    )
    def gather_kernel(p2_hbm, idx_hbm, out_hbm, idx_v, rows_v, sem):
        wid = lax.axis_index("s") * NC + lax.axis_index("c")
        base = wid * B_PER_W
        pltpu.sync_copy(idx_hbm.at[wid], idx_v)
        copies = []
        for j in range(NCHUNK):
            copies.append(
                pltpu.async_copy(
                    p2_hbm.at[idx_v.at[j]],
                    rows_v.at[pl.ds(j * CHUNK, CHUNK)],
                    sem,
                )
            )
        for c in copies:
            c.wait()
        pltpu.sync_copy(rows_v, out_hbm.at[pl.ds(base, B_PER_W)])

    return gather_kernel(p2, idx3)


_SROWS = 2048


def _sel_body(q_ref, m_ref, out_ref):
    take_left = m_ref[...] < S
    out_ref[...] = jnp.where(take_left, q_ref[:, 0:H], q_ref[:, H:2 * H])


def _tc_select(q, idxcol):
    grid = (B // _SROWS,)
    return pl.pallas_call(
        _sel_body,
        grid=grid,
        in_specs=[
            pl.BlockSpec((_SROWS, 2 * H), lambda i: (i, 0)),
            pl.BlockSpec((_SROWS, 1), lambda i: (i, 0)),
        ],
        out_specs=pl.BlockSpec((_SROWS, H), lambda i: (i, 0)),
        out_shape=jax.ShapeDtypeStruct((B, H), jnp.float32),
    )(q, idxcol)


def kernel(item_embeds, table, W, b):
    idx = item_embeds.astype(jnp.int32)
    j = jnp.where(idx < S, idx, idx - S)
    idx3 = j.reshape(NW, NCHUNK, CHUNK)
    tT = table.T
    p2 = _tc_project(tT, W, b.reshape(1, H))
    q = _sc_gather(p2, idx3)
    return _tc_select(q, idx.reshape(B, 1))
